# Initial kernel scaffold; baseline (speedup 1.0000x reference)
import jax
import jax.numpy as jnp
from jax.experimental import pallas as pl


def kernel(queries, memory):
    raise NotImplementedError("write your pallas kernel here")


# speedup vs baseline: 1.5247x; 1.5247x over previous
import functools

import jax
import jax.numpy as jnp
from jax.experimental import pallas as pl
from jax.experimental.pallas import tpu as pltpu

EPS = 1e-12
Q = 784
D = 512
K = 16384
KT = 2048
KT2 = 4096


def _knn_kernel(q_ref, m_ref, minval_ref, sidx_ref, star_ref, sstar_ref,
                mind2_ref, amin_ref):
    t = pl.program_id(0)
    nt = pl.num_programs(0)
    q = q_ref[...]
    m = m_ref[...]
    qm = jax.lax.dot_general(
        q, m, (((1,), (1,)), ((), ())),
        preferred_element_type=jnp.float32)
    q2 = jnp.sum(q * q, axis=1, keepdims=True)
    m2 = jnp.sum(m * m, axis=1)
    d2 = jnp.maximum(q2 + m2[None, :] - 2.0 * qm, 0.0)
    rowmin = jnp.min(d2, axis=1, keepdims=True)
    lanes = jax.lax.broadcasted_iota(jnp.int32, d2.shape, 1)
    rowarg = jnp.min(jnp.where(d2 == rowmin, lanes, K),
                     axis=1, keepdims=True) + t * KT

    @pl.when(t == 0)
    def _init():
        mind2_ref[...] = rowmin
        amin_ref[...] = rowarg

    @pl.when(t > 0)
    def _update():
        better = rowmin < mind2_ref[...]
        amin_ref[...] = jnp.where(better, rowarg, amin_ref[...])
        mind2_ref[...] = jnp.where(better, rowmin, mind2_ref[...])

    @pl.when(t == nt - 1)
    def _finalize():
        mv2 = mind2_ref[...]
        minval_ref[...] = jnp.sqrt(mv2 + EPS)
        smax = jnp.max(mv2)
        rows = jax.lax.broadcasted_iota(jnp.int32, mv2.shape, 0)
        sidx = jnp.min(jnp.where(mv2 == smax, rows, Q))
        star = jnp.min(jnp.where(rows == sidx, amin_ref[...], K))
        sidx_ref[0, 0] = sidx
        star_ref[0, 0] = star
        sstar_ref[0, 0] = jnp.sqrt(smax + EPS)


def _reweight_kernel(m_ref, mstar_ref, mtest_ref, sstar_ref, s_ref,
                     dstar_ref, dtest_ref):
    t = pl.program_id(0)
    nt = pl.num_programs(0)
    m = m_ref[...]
    ds = m - mstar_ref[...]
    dt = m - mtest_ref[...]
    dstar_ref[pl.ds(t * KT2, KT2), :] = jnp.sum(ds * ds, axis=1,
                                                keepdims=True)
    dtest_ref[pl.ds(t * KT2, KT2), :] = jnp.sum(dt * dt, axis=1,
                                                keepdims=True)

    @pl.when(t == nt - 1)
    def _finalize():
        wstar = dstar_ref[...]
        wtest = dtest_ref[...]
        rows = jax.lax.broadcasted_iota(jnp.int32, wstar.shape, 0)
        acc = 0.0
        cur = wstar
        for _ in range(3):
            mn = jnp.min(cur)
            idx = jnp.min(jnp.where(cur == mn, rows, K))
            dj2 = jnp.min(jnp.where(rows == idx, wtest, jnp.inf))
            acc = acc + jnp.exp(jnp.sqrt(dj2 + EPS))
            cur = jnp.where(rows == idx, jnp.inf, cur)
        s_star = sstar_ref[0, 0]
        s_ref[0, 0] = (1.0 - jnp.exp(s_star) / acc) * s_star


@functools.partial(jax.jit, static_argnums=())
def kernel(queries, memory):
    nt = K // KT
    minval, sidx, star, sstar = pl.pallas_call(
        _knn_kernel,
        grid=(nt,),
        in_specs=[
            pl.BlockSpec((Q, D), lambda t: (0, 0)),
            pl.BlockSpec((KT, D), lambda t: (t, 0)),
        ],
        out_specs=[
            pl.BlockSpec((Q, 1), lambda t: (0, 0)),
            pl.BlockSpec(memory_space=pltpu.SMEM),
            pl.BlockSpec(memory_space=pltpu.SMEM),
            pl.BlockSpec(memory_space=pltpu.SMEM),
        ],
        out_shape=[
            jax.ShapeDtypeStruct((Q, 1), jnp.float32),
            jax.ShapeDtypeStruct((1, 1), jnp.int32),
            jax.ShapeDtypeStruct((1, 1), jnp.int32),
            jax.ShapeDtypeStruct((1, 1), jnp.float32),
        ],
        scratch_shapes=[
            pltpu.VMEM((Q, 1), jnp.float32),
            pltpu.VMEM((Q, 1), jnp.int32),
        ],
    )(queries, memory)

    m_star = jnp.take(memory, star[0, 0], axis=0)[None, :]
    m_test = jnp.take(queries, sidx[0, 0], axis=0)[None, :]

    nt2 = K // KT2
    s = pl.pallas_call(
        _reweight_kernel,
        grid=(nt2,),
        in_specs=[
            pl.BlockSpec((KT2, D), lambda t: (t, 0)),
            pl.BlockSpec((1, D), lambda t: (0, 0)),
            pl.BlockSpec((1, D), lambda t: (0, 0)),
            pl.BlockSpec(memory_space=pltpu.SMEM),
        ],
        out_specs=pl.BlockSpec(memory_space=pltpu.SMEM),
        out_shape=jax.ShapeDtypeStruct((1, 1), jnp.float32),
        scratch_shapes=[
            pltpu.VMEM((K, 1), jnp.float32),
            pltpu.VMEM((K, 1), jnp.float32),
        ],
    )(memory, m_star, m_test, sstar)

    return (s[0, 0], minval.reshape(Q))


# speedup vs baseline: 1.9310x; 1.2665x over previous
import functools

import jax
import jax.numpy as jnp
from jax.experimental import pallas as pl
from jax.experimental.pallas import tpu as pltpu

EPS = 1e-12
Q = 784
D = 512
K = 16384
KT = 2048
KT2 = 4096


def _knn_kernel(q_ref, m_ref, minval_ref, sidx_ref, star_ref, sstar_ref,
                qs_ref, ming_ref, amin_ref):
    t = pl.program_id(0)
    nt = pl.num_programs(0)

    @pl.when(t == 0)
    def _prescale():
        qs_ref[...] = -2.0 * q_ref[...]

    m = m_ref[...]
    qm = jax.lax.dot_general(
        qs_ref[...], m, (((1,), (1,)), ((), ())),
        preferred_element_type=jnp.float32)
    m2 = jnp.sum(m * m, axis=1)
    g = qm + m2[None, :]
    rowmin = jnp.min(g, axis=1, keepdims=True)
    lanes = jax.lax.broadcasted_iota(jnp.int32, g.shape, 1)
    rowarg = jnp.min(jnp.where(g == rowmin, lanes, K),
                     axis=1, keepdims=True) + t * KT

    @pl.when(t == 0)
    def _init():
        ming_ref[...] = rowmin
        amin_ref[...] = rowarg

    @pl.when(t > 0)
    def _update():
        better = rowmin < ming_ref[...]
        amin_ref[...] = jnp.where(better, rowarg, amin_ref[...])
        ming_ref[...] = jnp.where(better, rowmin, ming_ref[...])

    @pl.when(t == nt - 1)
    def _finalize():
        q = q_ref[...]
        q2 = jnp.sum(q * q, axis=1, keepdims=True)
        mv2 = jnp.maximum(ming_ref[...] + q2, 0.0)
        minval_ref[...] = jnp.sqrt(mv2 + EPS)
        smax = jnp.max(mv2)
        rows = jax.lax.broadcasted_iota(jnp.int32, mv2.shape, 0)
        sidx = jnp.min(jnp.where(mv2 == smax, rows, Q))
        star = jnp.min(jnp.where(rows == sidx, amin_ref[...], K))
        sidx_ref[0, 0] = sidx
        star_ref[0, 0] = star
        sstar_ref[0, 0] = jnp.sqrt(smax + EPS)


def _reweight_kernel(m_ref, ms_ref, sstar_ref, s_ref, wstar_ref, wtest_ref):
    t = pl.program_id(0)
    nt = pl.num_programs(0)
    m = m_ref[...]
    m2 = jax.lax.dot_general(
        jnp.ones((1, D), jnp.float32), m * m, (((1,), (1,)), ((), ())),
        preferred_element_type=jnp.float32)
    cross = jax.lax.dot_general(
        ms_ref[...], m, (((1,), (1,)), ((), ())),
        preferred_element_type=jnp.float32)
    wstar_ref[pl.ds(t, 1), :] = m2 - 2.0 * cross[0:1, :]
    wtest_ref[pl.ds(t, 1), :] = m2 - 2.0 * cross[1:2, :]

    @pl.when(t == nt - 1)
    def _finalize():
        ms = ms_ref[...]
        s2 = jnp.sum(ms[0:1, :] * ms[0:1, :])
        t2 = jnp.sum(ms[1:2, :] * ms[1:2, :])
        ws = jnp.maximum(wstar_ref[...] + s2, 0.0)
        wt = jnp.maximum(wtest_ref[...] + t2, 0.0)
        rows = jax.lax.broadcasted_iota(jnp.int32, ws.shape, 0)
        lanes = jax.lax.broadcasted_iota(jnp.int32, ws.shape, 1)
        flat = rows * KT2 + lanes
        acc = 0.0
        for _ in range(3):
            mn = jnp.min(ws)
            idx = jnp.min(jnp.where(ws == mn, flat, K))
            dj2 = jnp.min(jnp.where(flat == idx, wt, jnp.inf))
            acc = acc + jnp.exp(jnp.sqrt(dj2 + EPS))
            ws = jnp.where(flat == idx, jnp.inf, ws)
        s_star = sstar_ref[0, 0]
        s_ref[0, 0] = (1.0 - jnp.exp(s_star) / acc) * s_star


@functools.partial(jax.jit, static_argnums=())
def kernel(queries, memory):
    nt = K // KT
    minval, sidx, star, sstar = pl.pallas_call(
        _knn_kernel,
        grid=(nt,),
        in_specs=[
            pl.BlockSpec((Q, D), lambda t: (0, 0)),
            pl.BlockSpec((KT, D), lambda t: (t, 0)),
        ],
        out_specs=[
            pl.BlockSpec((Q, 1), lambda t: (0, 0)),
            pl.BlockSpec(memory_space=pltpu.SMEM),
            pl.BlockSpec(memory_space=pltpu.SMEM),
            pl.BlockSpec(memory_space=pltpu.SMEM),
        ],
        out_shape=[
            jax.ShapeDtypeStruct((Q, 1), jnp.float32),
            jax.ShapeDtypeStruct((1, 1), jnp.int32),
            jax.ShapeDtypeStruct((1, 1), jnp.int32),
            jax.ShapeDtypeStruct((1, 1), jnp.float32),
        ],
        scratch_shapes=[
            pltpu.VMEM((Q, D), jnp.float32),
            pltpu.VMEM((Q, 1), jnp.float32),
            pltpu.VMEM((Q, 1), jnp.int32),
        ],
    )(queries, memory)

    m_star = jnp.take(memory, star[0, 0], axis=0)[None, :]
    m_test = jnp.take(queries, sidx[0, 0], axis=0)[None, :]
    ms = jnp.concatenate([m_star, m_test], axis=0)

    nt2 = K // KT2
    s = pl.pallas_call(
        _reweight_kernel,
        grid=(nt2,),
        in_specs=[
            pl.BlockSpec((KT2, D), lambda t: (t, 0)),
            pl.BlockSpec((2, D), lambda t: (0, 0)),
            pl.BlockSpec(memory_space=pltpu.SMEM),
        ],
        out_specs=pl.BlockSpec(memory_space=pltpu.SMEM),
        out_shape=jax.ShapeDtypeStruct((1, 1), jnp.float32),
        scratch_shapes=[
            pltpu.VMEM((nt2, KT2), jnp.float32),
            pltpu.VMEM((nt2, KT2), jnp.float32),
        ],
    )(memory, ms, sstar)

    return (s[0, 0], minval.reshape(Q))
